# baseline (speedup 1.0000x reference)
import jax
import jax.numpy as jnp
from jax import lax
from jax.experimental import pallas as pl
from jax.experimental.pallas import tpu as pltpu
from jax.experimental.pallas import tpu_sc as plsc

D = 64
SCALE = 8.0
NC = 2
NS = 16
NW = NC * NS
CH = 128


def _emb_body(idx_hbm, lut_hbm, out_hbm, idx_v, rows_v, gsem):
    wid = lax.axis_index("s") * NC + lax.axis_index("c")
    b_per_w = idx_hbm.shape[0] // NW
    base = wid * b_per_w
    pltpu.sync_copy(idx_hbm.at[pl.ds(base, b_per_w)], idx_v)
    n_chunks = b_per_w // CH

    @pl.loop(0, n_chunks)
    def _chunk(c):
        off = c * CH
        pltpu.async_copy(lut_hbm.at[idx_v.at[pl.ds(off, CH)]], rows_v,
                         gsem).wait()

        @pl.loop(0, CH)
        def _mul(r):
            for j in range(D // 16):
                rows_v[r, pl.ds(j * 16, 16)] = (
                    rows_v[r, pl.ds(j * 16, 16)] * SCALE)

        pltpu.sync_copy(rows_v, out_hbm.at[pl.ds(base + off, CH)])


def kernel(x, lut):
    b0, s = x.shape
    b = b0 * s
    idx = x.reshape(b).astype(jnp.int32)
    b_per_w = b // NW
    mesh = plsc.VectorSubcoreMesh(core_axis_name="c", subcore_axis_name="s")
    out_flat = pl.kernel(
        _emb_body,
        out_type=jax.ShapeDtypeStruct((b, D), jnp.float32),
        mesh=mesh,
        compiler_params=pltpu.CompilerParams(use_tc_tiling_on_sc=False),
        scratch_types=[
            pltpu.VMEM((b_per_w,), jnp.int32),
            pltpu.VMEM((CH, D), jnp.float32),
            pltpu.SemaphoreType.DMA,
        ],
    )(idx, lut)
    return out_flat.reshape(b0, s, D)


# speedup vs baseline: 1.5229x; 1.5229x over previous
import jax
import jax.numpy as jnp
from jax import lax
from jax.experimental import pallas as pl
from jax.experimental.pallas import tpu as pltpu
from jax.experimental.pallas import tpu_sc as plsc

D = 64
SCALE = 8.0
NC = 2
NS = 16
NW = NC * NS
S = 50
BB = 4096
B_PER_W = BB // NW
CB = 8
CR = CB * S
NG = CR // 16
NCH = B_PER_W // CB


def _emb_body(idx_hbm, lut_hbm, out_hbm, idx_v, rows_v, gsem):
    wid = lax.axis_index("s") * NC + lax.axis_index("c")
    base = wid * (B_PER_W * S)
    b0 = wid * B_PER_W
    pltpu.sync_copy(idx_hbm.at[pl.ds(base, B_PER_W * S)], idx_v)

    @pl.loop(0, NCH)
    def _chunk(c):
        c_off = c * CR

        @pl.loop(0, NG)
        def _fire(g):
            q = g * 16
            v = idx_v[pl.ds(c_off + q, 16)]
            for lane in range(16):
                r = v[lane]
                pltpu.async_copy(lut_hbm.at[pl.ds(r, 1), :],
                                 rows_v.at[pl.ds(q + lane, 1), :], gsem)

        pltpu.make_async_copy(lut_hbm.at[pl.ds(0, CR), :], rows_v,
                              gsem).wait()

        @pl.loop(0, CR)
        def _mul(s):
            for j in range(D // 16):
                rows_v[s, pl.ds(j * 16, 16)] = (
                    rows_v[s, pl.ds(j * 16, 16)] * SCALE)

        for k in range(CB):
            pltpu.sync_copy(rows_v.at[pl.ds(k * S, S), :],
                            out_hbm.at[b0 + c * CB + k])


def kernel(x, lut):
    b0, s = x.shape
    idx = x.reshape(b0 * s).astype(jnp.int32)
    mesh = plsc.VectorSubcoreMesh(core_axis_name="c", subcore_axis_name="s")
    out = pl.kernel(
        _emb_body,
        out_type=jax.ShapeDtypeStruct((b0, s, D), jnp.float32),
        mesh=mesh,
        scratch_types=[
            pltpu.VMEM((B_PER_W * S,), jnp.int32),
            pltpu.VMEM((CR, D), jnp.float32),
            pltpu.SemaphoreType.DMA,
        ],
    )(idx, lut)
    return out


# speedup vs baseline: 1.6559x; 1.0873x over previous
import jax
import jax.numpy as jnp
from jax import lax
from jax.experimental import pallas as pl
from jax.experimental.pallas import tpu as pltpu
from jax.experimental.pallas import tpu_sc as plsc

D = 64
SCALE = 8.0
NC = 2
NS = 16
NW = NC * NS
S = 50
BB = 4096
B_PER_W = BB // NW
CB = 8
CR = CB * S
NG = CR // 16
NCH = B_PER_W // CB


def _emb_body(idx_hbm, lut_hbm, out_hbm, idx_v,
              rows_a, rows_b, gsem_a, gsem_b, osem_a, osem_b):
    wid = lax.axis_index("s") * NC + lax.axis_index("c")
    base = wid * (B_PER_W * S)
    b0 = wid * B_PER_W
    pltpu.sync_copy(idx_hbm.at[pl.ds(base, B_PER_W * S)], idx_v)

    def fire(c, buf, gsem):
        @pl.loop(0, NG)
        def _fire(g):
            q = g * 16
            v = idx_v[pl.ds(c * CR + q, 16)]
            for lane in range(16):
                r = v[lane]
                pltpu.async_copy(lut_hbm.at[pl.ds(r, 1), :],
                                 buf.at[pl.ds(q + lane, 1), :], gsem)

    def drain_gather(buf, gsem):
        pltpu.make_async_copy(lut_hbm.at[pl.ds(0, CR), :], buf, gsem).wait()

    def scale(buf):
        @pl.loop(0, CR, unroll=4)
        def _mul(s):
            for j in range(D // 16):
                buf[s, pl.ds(j * 16, 16)] = buf[s, pl.ds(j * 16, 16)] * SCALE

    def out_start(c, buf, osem):
        for k in range(CB):
            pltpu.async_copy(buf.at[pl.ds(k * S, S), :],
                             out_hbm.at[b0 + c * CB + k], osem)

    def drain_out(buf, osem):
        pltpu.make_async_copy(lut_hbm.at[pl.ds(0, CR), :], buf, osem).wait()

    fire(0, rows_a, gsem_a)

    @pl.loop(0, NCH, step=2)
    def _pipe(c):
        @pl.when(c > 0)
        def _():
            drain_out(rows_b, osem_b)
        fire(c + 1, rows_b, gsem_b)
        drain_gather(rows_a, gsem_a)
        scale(rows_a)
        out_start(c, rows_a, osem_a)

        drain_out(rows_a, osem_a)

        @pl.when(c + 2 < NCH)
        def _():
            fire(c + 2, rows_a, gsem_a)
        drain_gather(rows_b, gsem_b)
        scale(rows_b)
        out_start(c + 1, rows_b, osem_b)

    drain_out(rows_b, osem_b)


def kernel(x, lut):
    b0, s = x.shape
    idx = x.reshape(b0 * s).astype(jnp.int32)
    mesh = plsc.VectorSubcoreMesh(core_axis_name="c", subcore_axis_name="s")
    out = pl.kernel(
        _emb_body,
        out_type=jax.ShapeDtypeStruct((b0, s, D), jnp.float32),
        mesh=mesh,
        scratch_types=[
            pltpu.VMEM((B_PER_W * S,), jnp.int32),
            pltpu.VMEM((CR, D), jnp.float32),
            pltpu.VMEM((CR, D), jnp.float32),
            pltpu.SemaphoreType.DMA,
            pltpu.SemaphoreType.DMA,
            pltpu.SemaphoreType.DMA,
            pltpu.SemaphoreType.DMA,
        ],
    )(idx, lut)
    return out
